# Initial kernel scaffold; baseline (speedup 1.0000x reference)
import jax
import jax.numpy as jnp
from jax.experimental import pallas as pl


def kernel(x, edge_index, edge_attr, batch, params):
    raise NotImplementedError("write your pallas kernel here")


# speedup vs baseline: 1.3886x; 1.3886x over previous
import functools

import jax
import jax.numpy as jnp
from jax import lax
from jax.experimental import pallas as pl
from jax.experimental.pallas import tpu as pltpu
from jax.experimental.pallas import tpu_sc as plsc

N = 50000
E = 800000
IN_DIM = 9
EDGE_DIM = 3
HID = 64
OUT_DIM = 128
NUM_GRAPHS = 512
EPS_BN = 1e-5

NC = 2
NS = 16
CHUNK = 128
EDGES_PER_TILE = 50176
E_PAD = NS * EDGES_PER_TILE
N_CHUNKS = EDGES_PER_TILE // CHUNK
HALF = N // NC
DUMP = HALF
SP_ROWS = 25088
ZB = 112

BN_ROWS = 2000
NB = N // BN_ROWS
BE_ROWS = 6272
NBE = E_PAD // BE_ROWS


def _aggr_sc_body(h_hbm, e_hbm, src_hbm, dst_hbm, out_hbm,
                  aggr_sp, src_v, dst_v, loc_v, hrows_v, erows_v, zbuf_v, sem):
    c = lax.axis_index("c")
    s = lax.axis_index("s")

    def _zb(j, carry):
        for k in range(HID // 16):
            zbuf_v[j, pl.ds(k * 16, 16)] = jnp.zeros((16,), jnp.float32)
        return carry
    lax.fori_loop(0, ZB, _zb, 0)

    def _zi(i, carry):
        off = pl.multiple_of((s * 14 + i) * ZB, 8)
        pltpu.sync_copy(zbuf_v, aggr_sp.at[pl.ds(off, ZB)])
        return carry
    lax.fori_loop(0, 14, _zi, 0)
    plsc.subcore_barrier()

    lo = c * HALF
    tile_base = s * EDGES_PER_TILE

    def _chunk(i, carry):
        base = pl.multiple_of(tile_base + i * CHUNK, 8)
        pltpu.sync_copy(src_hbm.at[pl.ds(base, CHUNK)], src_v)
        pltpu.sync_copy(dst_hbm.at[pl.ds(base, CHUNK)], dst_v)
        pltpu.async_copy(h_hbm.at[src_v], hrows_v, sem).wait()
        pltpu.sync_copy(e_hbm.at[pl.ds(base, CHUNK)], erows_v)

        def _row(j, rc):
            for k in range(HID // 16):
                hv = hrows_v[j, pl.ds(k * 16, 16)]
                ev = erows_v[j, pl.ds(k * 16, 16)]
                hrows_v[j, pl.ds(k * 16, 16)] = jnp.maximum(hv + ev, 0.0)
            return rc
        lax.fori_loop(0, CHUNK, _row, 0)

        def _idx(v, ic):
            d = dst_v[pl.ds(v * 16, 16)]
            l = d - lo
            ok = (l >= 0) & (l < HALF)
            loc_v[pl.ds(v * 16, 16)] = jnp.where(ok, l, DUMP)
            return ic
        lax.fori_loop(0, CHUNK // 16, _idx, 0)

        pltpu.sync_copy(hrows_v, aggr_sp.at[loc_v], add=True)
        return carry
    lax.fori_loop(0, N_CHUNKS, _chunk, 0)

    plsc.subcore_barrier()

    def _co(k, carry):
        cid = s + k * NS
        @pl.when(cid < HALF // 1000)
        def _():
            off = pl.multiple_of(cid * 1000, 8)
            pltpu.sync_copy(aggr_sp.at[pl.ds(off, 1000)],
                            out_hbm.at[pl.ds(lo + off, 1000)])
        return carry
    lax.fori_loop(0, 2, _co, 0)


_aggr_call = pl.kernel(
    _aggr_sc_body,
    out_type=jax.ShapeDtypeStruct((N, HID), jnp.float32),
    mesh=plsc.VectorSubcoreMesh(core_axis_name="c", subcore_axis_name="s"),
    scratch_types=[
        pltpu.VMEM_SHARED((SP_ROWS, HID), jnp.float32),
        pltpu.VMEM((CHUNK,), jnp.int32),
        pltpu.VMEM((CHUNK,), jnp.int32),
        pltpu.VMEM((CHUNK,), jnp.int32),
        pltpu.VMEM((CHUNK, HID), jnp.float32),
        pltpu.VMEM((CHUNK, HID), jnp.float32),
        pltpu.VMEM((ZB, HID), jnp.float32),
        pltpu.SemaphoreType.DMA,
    ],
    compiler_params=pltpu.CompilerParams(use_tc_tiling_on_sc=False),
)


def _enc_body(x_ref, w_ref, b_ref, o_ref):
    o_ref[...] = (jnp.dot(x_ref[...], w_ref[...],
                          preferred_element_type=jnp.float32) + b_ref[...])


def _lin_stats_body(h_ref, a_ref, w_ref, b_ref, y_ref, s_ref):
    i = pl.program_id(0)
    z = h_ref[...] + a_ref[...]
    y = jnp.dot(z, w_ref[...], preferred_element_type=jnp.float32) + b_ref[...]
    y_ref[...] = y

    @pl.when(i == 0)
    def _():
        s_ref[...] = jnp.zeros_like(s_ref)
    s_ref[0:1, :] += jnp.sum(y, axis=0, keepdims=True)
    s_ref[1:2, :] += jnp.sum(y * y, axis=0, keepdims=True)


def _bn_lin_stats_body(y_ref, s1_ref, g_ref, bb_ref, w_ref, b_ref,
                       u_ref, s2_ref):
    i = pl.program_id(0)
    s1 = s1_ref[...]
    mean = s1[0:1, :] * (1.0 / N)
    var = s1[1:2, :] * (1.0 / N) - mean * mean
    scale = lax.rsqrt(var + EPS_BN) * g_ref[...]
    a = jnp.maximum((y_ref[...] - mean) * scale + bb_ref[...], 0.0)
    u = jnp.dot(a, w_ref[...], preferred_element_type=jnp.float32) + b_ref[...]
    u_ref[...] = u

    @pl.when(i == 0)
    def _():
        s2_ref[...] = jnp.zeros_like(s2_ref)
    s2_ref[0:1, :] += jnp.sum(u, axis=0, keepdims=True)
    s2_ref[1:2, :] += jnp.sum(u * u, axis=0, keepdims=True)


def _bn_relu_body(u_ref, s_ref, g_ref, bb_ref, o_ref):
    s = s_ref[...]
    mean = s[0:1, :] * (1.0 / N)
    var = s[1:2, :] * (1.0 / N) - mean * mean
    scale = lax.rsqrt(var + EPS_BN) * g_ref[...]
    o_ref[...] = jnp.maximum((u_ref[...] - mean) * scale + bb_ref[...], 0.0)


def _bn_relu_pool_body(u_ref, s_ref, g_ref, bb_ref, batch_ref,
                       sums_ref, cnt_ref):
    i = pl.program_id(0)
    s = s_ref[...]
    mean = s[0:1, :] * (1.0 / N)
    var = s[1:2, :] * (1.0 / N) - mean * mean
    scale = lax.rsqrt(var + EPS_BN) * g_ref[...]
    h = jnp.maximum((u_ref[...] - mean) * scale + bb_ref[...], 0.0)

    gids = lax.broadcasted_iota(jnp.int32, (1, NUM_GRAPHS), 1)
    onehot = (batch_ref[...] == gids).astype(jnp.float32)

    @pl.when(i == 0)
    def _():
        sums_ref[...] = jnp.zeros_like(sums_ref)
        cnt_ref[...] = jnp.zeros_like(cnt_ref)
    sums_ref[...] += lax.dot_general(
        onehot, h, (((0,), (0,)), ((), ())),
        preferred_element_type=jnp.float32)
    cnt_ref[...] += jnp.sum(onehot, axis=0, keepdims=True).T


def _pool_proj_body(s_ref, c_ref, w_ref, b_ref, o_ref):
    pooled = s_ref[...] / jnp.maximum(c_ref[...], 1.0)
    o_ref[...] = (jnp.dot(pooled, w_ref[...],
                          preferred_element_type=jnp.float32) + b_ref[...])


def _row_spec(rows, cols):
    return pl.BlockSpec((rows, cols), lambda i: (i, 0))


def _const_spec(rows, cols):
    return pl.BlockSpec((rows, cols), lambda i: (0, 0))


_enc_atom = pl.pallas_call(
    _enc_body,
    grid=(NB,),
    in_specs=[_row_spec(BN_ROWS, 16), _const_spec(16, HID),
              _const_spec(1, HID)],
    out_specs=_row_spec(BN_ROWS, HID),
    out_shape=jax.ShapeDtypeStruct((N, HID), jnp.float32),
)

_enc_bond = pl.pallas_call(
    _enc_body,
    grid=(NBE,),
    in_specs=[_row_spec(BE_ROWS, 8), _const_spec(8, HID),
              _const_spec(1, HID)],
    out_specs=_row_spec(BE_ROWS, HID),
    out_shape=jax.ShapeDtypeStruct((E_PAD, HID), jnp.float32),
)

_lin_stats = pl.pallas_call(
    _lin_stats_body,
    grid=(NB,),
    in_specs=[_row_spec(BN_ROWS, HID), _row_spec(BN_ROWS, HID),
              _const_spec(HID, HID), _const_spec(1, HID)],
    out_specs=[_row_spec(BN_ROWS, HID), _const_spec(2, HID)],
    out_shape=[jax.ShapeDtypeStruct((N, HID), jnp.float32),
               jax.ShapeDtypeStruct((2, HID), jnp.float32)],
)

_bn_lin_stats = pl.pallas_call(
    _bn_lin_stats_body,
    grid=(NB,),
    in_specs=[_row_spec(BN_ROWS, HID), _const_spec(2, HID),
              _const_spec(1, HID), _const_spec(1, HID),
              _const_spec(HID, HID), _const_spec(1, HID)],
    out_specs=[_row_spec(BN_ROWS, HID), _const_spec(2, HID)],
    out_shape=[jax.ShapeDtypeStruct((N, HID), jnp.float32),
               jax.ShapeDtypeStruct((2, HID), jnp.float32)],
)

_bn_relu = pl.pallas_call(
    _bn_relu_body,
    grid=(NB,),
    in_specs=[_row_spec(BN_ROWS, HID), _const_spec(2, HID),
              _const_spec(1, HID), _const_spec(1, HID)],
    out_specs=_row_spec(BN_ROWS, HID),
    out_shape=jax.ShapeDtypeStruct((N, HID), jnp.float32),
)

_bn_relu_pool = pl.pallas_call(
    _bn_relu_pool_body,
    grid=(NB,),
    in_specs=[_row_spec(BN_ROWS, HID), _const_spec(2, HID),
              _const_spec(1, HID), _const_spec(1, HID),
              _row_spec(BN_ROWS, 1)],
    out_specs=[_const_spec(NUM_GRAPHS, HID), _const_spec(NUM_GRAPHS, 1)],
    out_shape=[jax.ShapeDtypeStruct((NUM_GRAPHS, HID), jnp.float32),
               jax.ShapeDtypeStruct((NUM_GRAPHS, 1), jnp.float32)],
)

_pool_proj = pl.pallas_call(
    _pool_proj_body,
    grid=(1,),
    in_specs=[_const_spec(NUM_GRAPHS, HID), _const_spec(NUM_GRAPHS, 1),
              _const_spec(HID, OUT_DIM), _const_spec(1, OUT_DIM)],
    out_specs=_const_spec(NUM_GRAPHS, OUT_DIM),
    out_shape=jax.ShapeDtypeStruct((NUM_GRAPHS, OUT_DIM), jnp.float32),
)


def kernel(x, edge_index, edge_attr, batch, params):
    p = params
    src = edge_index[0].astype(jnp.int32)
    dst = edge_index[1].astype(jnp.int32)
    src_pad = jnp.pad(src, (0, E_PAD - E))
    dst_pad = jnp.pad(dst, (0, E_PAD - E), constant_values=N)
    ea_pad = jnp.pad(edge_attr, ((0, E_PAD - E), (0, 8 - EDGE_DIM)))
    x_pad = jnp.pad(x, ((0, 0), (0, 16 - IN_DIM)))
    aw = jnp.pad(p["atom_W"], ((0, 16 - IN_DIM), (0, 0)))
    bw = jnp.pad(p["bond_W"], ((0, 8 - EDGE_DIM), (0, 0)))
    batch2d = batch.astype(jnp.int32).reshape(N, 1)

    def row(v):
        return v.reshape(1, -1)

    h = _enc_atom(x_pad, aw, row(p["atom_b"]))
    e = _enc_bond(ea_pad, bw, row(p["bond_b"]))

    sums = cnts = None
    for l, lp in enumerate(p["layers"]):
        aggr = _aggr_call(h, e, src_pad, dst_pad)
        y, s1 = _lin_stats(h, aggr, lp["lin1_W"], row(lp["lin1_b"]))
        u, s2 = _bn_lin_stats(y, s1, row(lp["bn1_g"]), row(lp["bn1_b"]),
                              lp["lin2_W"], row(lp["lin2_b"]))
        if l < 2:
            h = _bn_relu(u, s2, row(lp["bn_g"]), row(lp["bn_b"]))
        else:
            sums, cnts = _bn_relu_pool(u, s2, row(lp["bn_g"]),
                                       row(lp["bn_b"]), batch2d)

    return _pool_proj(sums, cnts, p["proj_W"], row(p["proj_b"]))
